# baseline (speedup 1.0000x reference)
import functools

import jax
import jax.numpy as jnp
from jax import lax
from jax.experimental import pallas as pl
from jax.experimental.pallas import tpu as pltpu
from jax.experimental.pallas import tpu_sc as plsc

N = 10000
E = 320000
D = 128
H = 128
OUT = 64
NG = 64

NPAD = 10112
CH = 128
NTILES = 16
NCORES = 2
NW = NCORES * NTILES
NCH = 80
EPW = NCH * CH
EPAD = NW * EPW
TOTCH = EPAD // CH
RPT = NPAD // NTILES
G = NCH // 2


NB = 256
_BMUL = 52429


def _sc_sort_body(src_hbm, perm_hbm, sin, perm2, cnt):
    c = lax.axis_index("c")
    s = lax.axis_index("s")
    w = c * NTILES + s
    e0 = w * EPW
    base = w * NCH
    pltpu.sync_copy(src_hbm.at[pl.ds(e0, EPW)], sin)
    lane = lax.iota(jnp.int32, 16)

    def zero(i, carry):
        cnt[i] = 0
        return carry

    lax.fori_loop(0, NB, zero, 0)

    def hist(j, carry):
        bv = (sin[pl.ds(j * 16, 16)] * _BMUL) >> 21
        for k in range(16):
            b = bv[k]
            cnt[b] = cnt[b] + 1
        return carry

    lax.fori_loop(0, EPW // 16, hist, 0)

    def prefix(b, acc):
        nxt = acc + cnt[b]
        cnt[b] = acc
        return nxt

    lax.fori_loop(0, NB, prefix, 0)

    def pchunk(k, carry):
        for sub in range(8):
            vv = sin[pl.ds(k * CH + sub * 16, 16)]
            bv = (vv * _BMUL) >> 21
            pv = jnp.zeros((16,), jnp.int32)
            for kk in range(16):
                b = bv[kk]
                p = cnt[b]
                cnt[b] = p + 1
                pv = jnp.where(lane == kk, p, pv)
            perm2[k, pl.ds(sub * 16, 16)] = pv + e0
        return carry

    lax.fori_loop(0, NCH, pchunk, 0)
    pltpu.sync_copy(perm2, perm_hbm.at[pl.ds(base, NCH)])


_sc_sort = pl.kernel(
    _sc_sort_body,
    out_type=jax.ShapeDtypeStruct((TOTCH, CH), jnp.int32),
    mesh=plsc.VectorSubcoreMesh(core_axis_name="c", subcore_axis_name="s",
                                num_cores=NCORES, num_subcores=NTILES),
    scratch_types=[
        pltpu.VMEM((EPW,), jnp.int32),
        pltpu.VMEM((NCH, CH), jnp.int32),
        pltpu.SMEM((NB,), jnp.int32),
    ],
)


def _sc_agg_body(h_hbm, src_hbm, dst_hbm, z_hbm, out_hbm,
                 acc_sh, sidx, didx, rows_v, sem):
    c = lax.axis_index("c")
    s = lax.axis_index("s")
    w = c * NTILES + s
    base = w * NCH
    pltpu.sync_copy(z_hbm.at[pl.ds(s * RPT, RPT)],
                    acc_sh.at[pl.ds(s * RPT, RPT)])
    plsc.subcore_barrier()

    def chunk(i, carry):
        off = (base + i) * CH
        pltpu.sync_copy(src_hbm.at[pl.ds(off, CH)], sidx)
        pltpu.sync_copy(dst_hbm.at[pl.ds(off, CH)], didx)
        pltpu.async_copy(h_hbm.at[sidx], rows_v, sem).wait()
        pltpu.sync_copy(rows_v, acc_sh.at[didx], add=True)
        return carry

    lax.fori_loop(0, NCH, chunk, 0)
    plsc.subcore_barrier()
    pltpu.sync_copy(acc_sh.at[pl.ds(s * RPT, RPT)],
                    out_hbm.at[c, pl.ds(s * RPT, RPT)])


_sc_agg = pl.kernel(
    _sc_agg_body,
    out_type=jax.ShapeDtypeStruct((NCORES, NPAD, D), jnp.float32),
    mesh=plsc.VectorSubcoreMesh(core_axis_name="c", subcore_axis_name="s",
                                num_cores=NCORES, num_subcores=NTILES),
    scratch_types=[
        pltpu.VMEM_SHARED((NPAD, D), jnp.float32),
        pltpu.VMEM((CH,), jnp.int32),
        pltpu.VMEM((CH,), jnp.int32),
        pltpu.VMEM((CH, D), jnp.float32),
        pltpu.SemaphoreType.DMA,
    ],
)


def _layer_body(hprev, agg2, batch, W1, b1, W2, b2, g, be, hout, pool):
    h_in = hprev[...] + agg2[0] + agg2[1]
    t = jnp.dot(h_in, W1[...], preferred_element_type=jnp.float32) + b1[...]
    t = jnp.maximum(t, 0.0)
    t = jnp.dot(t, W2[...], preferred_element_type=jnp.float32) + b2[...]
    mask = (lax.broadcasted_iota(jnp.int32, (NPAD, 1), 0) < N).astype(
        jnp.float32)
    m = jnp.sum(t * mask, axis=0, keepdims=True) / N
    v = jnp.sum((t - m) ** 2 * mask, axis=0, keepdims=True) / N
    h = (t - m) * lax.rsqrt(v + 1e-5) * g[...] + be[...]
    h = jnp.maximum(h, 0.0) * mask
    hout[...] = h
    oh = (batch[...] == lax.broadcasted_iota(jnp.int32, (NPAD, NG), 1)
          ).astype(jnp.float32)
    seg = lax.dot_general(oh, h, (((0,), (0,)), ((), ())),
                          preferred_element_type=jnp.float32)
    cnt = jnp.sum(oh, axis=0)
    pool[...] = seg / jnp.maximum(cnt, 1.0)[:, None]


_layer = pl.pallas_call(
    _layer_body,
    out_shape=(
        jax.ShapeDtypeStruct((NPAD, H), jnp.float32),
        jax.ShapeDtypeStruct((NG, H), jnp.float32),
    ),
)


def _final_body(p1, p2, p3, Wl, bl, out):
    hf = jnp.concatenate([p1[...], p2[...], p3[...]], axis=1)
    o = jnp.dot(hf, Wl[...], preferred_element_type=jnp.float32) + bl[...]
    nrm = jnp.sqrt(jnp.sum(o * o, axis=1, keepdims=True))
    out[...] = o / jnp.maximum(nrm, 1e-12)


_final = pl.pallas_call(
    _final_body,
    out_shape=jax.ShapeDtypeStruct((NG, OUT), jnp.float32),
)


def kernel(x, edge_index, batch, W1_1, b1_1, W2_1, b2_1, g_1, be_1,
           W1_2, b1_2, W2_2, b2_2, g_2, be_2, W1_3, b1_3, W2_3, b2_3,
           g_3, be_3, Wlin, blin):
    xp = jnp.zeros((NPAD, D), jnp.float32).at[:N].set(x)
    pad_e = EPAD - E
    src = jnp.concatenate(
        [edge_index[0], jnp.full((pad_e,), NPAD - 1, jnp.int32)])
    dst = jnp.concatenate(
        [edge_index[1], jnp.full((pad_e,), NPAD - 1, jnp.int32)])
    perm = _sc_sort(src).reshape(EPAD)
    src = jnp.zeros((EPAD,), jnp.int32).at[perm].set(
        src, unique_indices=True, mode="promise_in_bounds")
    dst = jnp.zeros((EPAD,), jnp.int32).at[perm].set(
        dst, unique_indices=True, mode="promise_in_bounds")
    bat = jnp.full((NPAD, 1), NG, jnp.int32).at[:N, 0].set(batch)
    z = jnp.zeros((NPAD, D), jnp.float32)

    h = xp
    pools = []
    for (W1, b1, W2, b2, g, be) in (
            (W1_1, b1_1, W2_1, b2_1, g_1, be_1),
            (W1_2, b1_2, W2_2, b2_2, g_2, be_2),
            (W1_3, b1_3, W2_3, b2_3, g_3, be_3)):
        agg2 = _sc_agg(h, src, dst, z)
        h, pool = _layer(h, agg2, bat, W1, b1, W2, b2, g, be)
        pools.append(pool)

    return _final(pools[0], pools[1], pools[2], Wlin, blin)


# speedup vs baseline: 2.4606x; 2.4606x over previous
import functools

import jax
import jax.numpy as jnp
from jax import lax
from jax.experimental import pallas as pl
from jax.experimental.pallas import tpu as pltpu
from jax.experimental.pallas import tpu_sc as plsc

N = 10000
E = 320000
D = 128
H = 128
OUT = 64
NG = 64

NPAD = 10112
CH = 128
NTILES = 16
NCORES = 2
NW = NCORES * NTILES
NCH = 80
EPW = NCH * CH
EPAD = NW * EPW
TOTCH = EPAD // CH
RPT = NPAD // NTILES
G = NCH // 2


def _sc_agg_body(h_hbm, src_hbm, dst_hbm, z_hbm, out_hbm,
                 acc_sh, sidx, didx, rows_v, sem):
    c = lax.axis_index("c")
    s = lax.axis_index("s")
    w = c * NTILES + s
    base = w * NCH
    pltpu.sync_copy(z_hbm.at[pl.ds(s * RPT, RPT)],
                    acc_sh.at[pl.ds(s * RPT, RPT)])
    plsc.subcore_barrier()

    def chunk(i, carry):
        off = (base + i) * CH
        pltpu.sync_copy(src_hbm.at[pl.ds(off, CH)], sidx)
        pltpu.sync_copy(dst_hbm.at[pl.ds(off, CH)], didx)
        pltpu.async_copy(h_hbm.at[sidx], rows_v, sem).wait()
        pltpu.sync_copy(rows_v, acc_sh.at[didx], add=True)
        return carry

    lax.fori_loop(0, NCH, chunk, 0)
    plsc.subcore_barrier()
    pltpu.sync_copy(acc_sh.at[pl.ds(s * RPT, RPT)],
                    out_hbm.at[c, pl.ds(s * RPT, RPT)])


_sc_agg = pl.kernel(
    _sc_agg_body,
    out_type=jax.ShapeDtypeStruct((NCORES, NPAD, D), jnp.float32),
    mesh=plsc.VectorSubcoreMesh(core_axis_name="c", subcore_axis_name="s",
                                num_cores=NCORES, num_subcores=NTILES),
    scratch_types=[
        pltpu.VMEM_SHARED((NPAD, D), jnp.float32),
        pltpu.VMEM((CH,), jnp.int32),
        pltpu.VMEM((CH,), jnp.int32),
        pltpu.VMEM((CH, D), jnp.float32),
        pltpu.SemaphoreType.DMA,
    ],
)


def _layer_body(hprev, agg2, batch, W1, b1, W2, b2, g, be, hout, pool):
    h_in = hprev[...] + agg2[0] + agg2[1]
    t = jnp.dot(h_in, W1[...], preferred_element_type=jnp.float32) + b1[...]
    t = jnp.maximum(t, 0.0)
    t = jnp.dot(t, W2[...], preferred_element_type=jnp.float32) + b2[...]
    mask = (lax.broadcasted_iota(jnp.int32, (NPAD, 1), 0) < N).astype(
        jnp.float32)
    m = jnp.sum(t * mask, axis=0, keepdims=True) / N
    v = jnp.sum((t - m) ** 2 * mask, axis=0, keepdims=True) / N
    h = (t - m) * lax.rsqrt(v + 1e-5) * g[...] + be[...]
    h = jnp.maximum(h, 0.0) * mask
    hout[...] = h
    oh = (batch[...] == lax.broadcasted_iota(jnp.int32, (NPAD, NG), 1)
          ).astype(jnp.float32)
    seg = lax.dot_general(oh, h, (((0,), (0,)), ((), ())),
                          preferred_element_type=jnp.float32)
    cnt = jnp.sum(oh, axis=0)
    pool[...] = seg / jnp.maximum(cnt, 1.0)[:, None]


_layer = pl.pallas_call(
    _layer_body,
    out_shape=(
        jax.ShapeDtypeStruct((NPAD, H), jnp.float32),
        jax.ShapeDtypeStruct((NG, H), jnp.float32),
    ),
)


def _final_body(p1, p2, p3, Wl, bl, out):
    hf = jnp.concatenate([p1[...], p2[...], p3[...]], axis=1)
    o = jnp.dot(hf, Wl[...], preferred_element_type=jnp.float32) + bl[...]
    nrm = jnp.sqrt(jnp.sum(o * o, axis=1, keepdims=True))
    out[...] = o / jnp.maximum(nrm, 1e-12)


_final = pl.pallas_call(
    _final_body,
    out_shape=jax.ShapeDtypeStruct((NG, OUT), jnp.float32),
)


def kernel(x, edge_index, batch, W1_1, b1_1, W2_1, b2_1, g_1, be_1,
           W1_2, b1_2, W2_2, b2_2, g_2, be_2, W1_3, b1_3, W2_3, b2_3,
           g_3, be_3, Wlin, blin):
    xp = jnp.zeros((NPAD, D), jnp.float32).at[:N].set(x)
    pad_e = EPAD - E
    src = jnp.concatenate(
        [edge_index[0], jnp.full((pad_e,), NPAD - 1, jnp.int32)])
    dst = jnp.concatenate(
        [edge_index[1], jnp.full((pad_e,), NPAD - 1, jnp.int32)])
    bat = jnp.full((NPAD, 1), NG, jnp.int32).at[:N, 0].set(batch)
    z = jnp.zeros((NPAD, D), jnp.float32)

    h = xp
    pools = []
    for (W1, b1, W2, b2, g, be) in (
            (W1_1, b1_1, W2_1, b2_1, g_1, be_1),
            (W1_2, b1_2, W2_2, b2_2, g_2, be_2),
            (W1_3, b1_3, W2_3, b2_3, g_3, be_3)):
        agg2 = _sc_agg(h, src, dst, z)
        h, pool = _layer(h, agg2, bat, W1, b1, W2, b2, g, be)
        pools.append(pool)

    return _final(pools[0], pools[1], pools[2], Wlin, blin)


# speedup vs baseline: 3.5456x; 1.4409x over previous
import functools

import jax
import jax.numpy as jnp
from jax import lax
from jax.experimental import pallas as pl
from jax.experimental.pallas import tpu as pltpu
from jax.experimental.pallas import tpu_sc as plsc

N = 10000
E = 320000
D = 128
H = 128
OUT = 64
NG = 64

NPAD = 10112
CH = 128
NTILES = 16
NCORES = 2
NW = NCORES * NTILES
NCH = 79
EPW = NCH * CH
EPAD = NW * EPW
TOTCH = EPAD // CH
RPT = NPAD // NTILES
G = NCH // 2


def _sc_agg_body(h_hbm, src_hbm, dst_hbm, z_hbm, out_hbm,
                 acc_sh, sidx, didx, rows_v, sem):
    c = lax.axis_index("c")
    s = lax.axis_index("s")
    w = c * NTILES + s
    base = w * NCH
    pltpu.sync_copy(z_hbm.at[pl.ds(s * RPT, RPT)],
                    acc_sh.at[pl.ds(s * RPT, RPT)])
    plsc.subcore_barrier()

    def chunk(i, carry):
        off = (base + i) * CH
        pltpu.sync_copy(src_hbm.at[pl.ds(off, CH)], sidx)
        pltpu.sync_copy(dst_hbm.at[pl.ds(off, CH)], didx)
        pltpu.async_copy(h_hbm.at[sidx], rows_v, sem).wait()
        pltpu.sync_copy(rows_v, acc_sh.at[didx], add=True)
        return carry

    lax.fori_loop(0, NCH, chunk, 0)
    plsc.subcore_barrier()
    pltpu.sync_copy(acc_sh.at[pl.ds(s * RPT, RPT)],
                    out_hbm.at[c, pl.ds(s * RPT, RPT)])


_sc_agg = pl.kernel(
    _sc_agg_body,
    out_type=jax.ShapeDtypeStruct((NCORES, NPAD, D), jnp.float32),
    mesh=plsc.VectorSubcoreMesh(core_axis_name="c", subcore_axis_name="s",
                                num_cores=NCORES, num_subcores=NTILES),
    scratch_types=[
        pltpu.VMEM_SHARED((NPAD, D), jnp.float32),
        pltpu.VMEM((CH,), jnp.int32),
        pltpu.VMEM((CH,), jnp.int32),
        pltpu.VMEM((CH, D), jnp.float32),
        pltpu.SemaphoreType.DMA,
    ],
)


def _layer_body(hprev, agg2, batch, W1, b1, W2, b2, g, be, hout, pool):
    h_in = hprev[...] + agg2[0] + agg2[1]
    t = jnp.dot(h_in, W1[...], preferred_element_type=jnp.float32) + b1[...]
    t = jnp.maximum(t, 0.0)
    t = jnp.dot(t, W2[...], preferred_element_type=jnp.float32) + b2[...]
    mask = (lax.broadcasted_iota(jnp.int32, (NPAD, 1), 0) < N).astype(
        jnp.float32)
    m = jnp.sum(t * mask, axis=0, keepdims=True) / N
    v = jnp.sum((t - m) ** 2 * mask, axis=0, keepdims=True) / N
    h = (t - m) * lax.rsqrt(v + 1e-5) * g[...] + be[...]
    h = jnp.maximum(h, 0.0) * mask
    hout[...] = h
    oh = (batch[...] == lax.broadcasted_iota(jnp.int32, (NPAD, NG), 1)
          ).astype(jnp.float32)
    seg = lax.dot_general(oh, h, (((0,), (0,)), ((), ())),
                          preferred_element_type=jnp.float32)
    cnt = jnp.sum(oh, axis=0)
    pool[...] = seg / jnp.maximum(cnt, 1.0)[:, None]


_layer = pl.pallas_call(
    _layer_body,
    out_shape=(
        jax.ShapeDtypeStruct((NPAD, H), jnp.float32),
        jax.ShapeDtypeStruct((NG, H), jnp.float32),
    ),
)


def _final_body(p1, p2, p3, Wl, bl, out):
    hf = jnp.concatenate([p1[...], p2[...], p3[...]], axis=1)
    o = jnp.dot(hf, Wl[...], preferred_element_type=jnp.float32) + bl[...]
    nrm = jnp.sqrt(jnp.sum(o * o, axis=1, keepdims=True))
    out[...] = o / jnp.maximum(nrm, 1e-12)


_final = pl.pallas_call(
    _final_body,
    out_shape=jax.ShapeDtypeStruct((NG, OUT), jnp.float32),
)


def kernel(x, edge_index, batch, W1_1, b1_1, W2_1, b2_1, g_1, be_1,
           W1_2, b1_2, W2_2, b2_2, g_2, be_2, W1_3, b1_3, W2_3, b2_3,
           g_3, be_3, Wlin, blin):
    xp = jnp.zeros((NPAD, D), jnp.float32).at[:N].set(x)
    pad_e = EPAD - E
    src = jnp.concatenate(
        [edge_index[0], jnp.full((pad_e,), NPAD - 1, jnp.int32)])
    dst = jnp.concatenate(
        [edge_index[1], jnp.full((pad_e,), NPAD - 1, jnp.int32)])
    bat = jnp.full((NPAD, 1), NG, jnp.int32).at[:N, 0].set(batch)
    z = jnp.zeros((NPAD, D), jnp.float32)

    h = xp
    pools = []
    for (W1, b1, W2, b2, g, be) in (
            (W1_1, b1_1, W2_1, b2_1, g_1, be_1),
            (W1_2, b1_2, W2_2, b2_2, g_2, be_2),
            (W1_3, b1_3, W2_3, b2_3, g_3, be_3)):
        agg2 = _sc_agg(h, src, dst, z)
        h, pool = _layer(h, agg2, bat, W1, b1, W2, b2, g, be)
        pools.append(pool)

    return _final(pools[0], pools[1], pools[2], Wlin, blin)
